# baseline (speedup 1.0000x reference)
import functools

import jax
import jax.numpy as jnp
from jax import lax
from jax.experimental import pallas as pl
from jax.experimental.pallas import tpu as pltpu
from jax.experimental.pallas import tpu_sc as plsc

_NUM_CLASS = 1000
_B = 1024
_L = 50
_CB = 40
_NCC = _NUM_CLASS // _CB
_NTASK = _L * _NCC
_NW = 32
_SLOTS = 40
_NG = _B // 16

_mesh = plsc.VectorSubcoreMesh(core_axis_name="c", subcore_axis_name="s")


@functools.partial(
    pl.kernel,
    mesh=_mesh,
    compiler_params=pltpu.CompilerParams(
        needs_layout_passes=False,
        skip_device_barrier=True,
        disable_bounds_checks=True,
        disable_semaphore_checks=True,
    ),
    out_type=jax.ShapeDtypeStruct((_L, _NUM_CLASS, _B), jnp.float32),
    scratch_types=[
        pltpu.VMEM((_CB, _B), jnp.float32),
        pltpu.VMEM((_CB, _B), jnp.float32),
        pltpu.VMEM((_B,), jnp.int32),
        pltpu.VMEM((_B,), jnp.int32),
        pltpu.VMEM((_B,), jnp.int32),
        pltpu.VMEM((_B,), jnp.int32),
        pltpu.VMEM((_B,), jnp.int32),
        pltpu.VMEM((_B,), jnp.int32),
        pltpu.SemaphoreType.DMA,
        pltpu.SemaphoreType.DMA,
        pltpu.SemaphoreType.DMA,
        pltpu.SemaphoreType.DMA,
    ],
)
def _sc_onehot_t(xt_hbm, out_hbm, buf0, buf1, xc0, xc1, cl0, cl1, bl0, bl1,
                 sem0, sem1, semc0, semc1):
    wid = lax.axis_index("s") * 2 + lax.axis_index("c")
    lane = lax.broadcasted_iota(jnp.int32, (16,), 0)
    zeros16 = jnp.zeros((16,), jnp.float32)
    ones16 = jnp.ones((16,), jnp.float32)

    bufs = (buf0, buf1)
    xcols = (xc0, xc1)
    clogs = (cl0, cl1)
    blogs = (bl0, bl1)
    sems = (sem0, sem1)
    semcs = (semc0, semc1)

    def _advance(l, cc):
        cc = cc + (_NW - _NCC)
        over = cc >= _NCC
        return l + 1 + over.astype(jnp.int32), jnp.where(over, cc - _NCC, cc)

    l_init = (wid >= _NCC).astype(jnp.int32)
    cc_init = jnp.where(wid >= _NCC, wid - _NCC, wid)

    def _zero_row(r, carry):
        for buf in bufs:
            for g in range(_NG):
                buf[r, pl.ds(16 * g, 16)] = zeros16
        return carry

    lax.fori_loop(0, _CB, _zero_row, 0)

    zi16 = jnp.zeros((16,), jnp.int32)
    for g in range(_NG):
        for log in (cl0, cl1, bl0, bl1):
            log[pl.ds(16 * g, 16)] = zi16

    pltpu.async_copy(xt_hbm.at[l_init], xc0, semc0)

    def _plant(p, c0):
        buf, xcol, clog, blog = bufs[p], xcols[p], clogs[p], blogs[p]
        for g in range(_NG):
            xv = xcol[pl.ds(16 * g, 16)]
            c_rel = xv - c0
            m = (c_rel >= 0) & (c_rel < _CB)
            bv = 16 * g + lane
            plsc.store_scatter(buf, [c_rel, bv], ones16, mask=m)
            plsc.store_compressed(clog.at[pl.ds(16 * g, 16)], c_rel, mask=m)
            plsc.store_compressed(blog.at[pl.ds(16 * g, 16)], bv, mask=m)

    def _clear(p):
        buf, clog, blog = bufs[p], clogs[p], blogs[p]
        for g in range(_NG):
            cidx = clog[pl.ds(16 * g, 16)]
            bidx = blog[pl.ds(16 * g, 16)]
            plsc.store_scatter(buf, [cidx, bidx], zeros16)

    def _slot_body(s_pair, carry):
        l, cc, infl = carry
        infl = list(infl)
        for p in range(2):
            s = 2 * s_pair + p
            t = wid + _NW * s
            c0 = pl.multiple_of(cc * _CB, _CB)

            l_n, _ = _advance(l, cc)

            @pl.when((s + 1 < _SLOTS) & (t + _NW < _NTASK))
            def _():
                pltpu.async_copy(xt_hbm.at[l_n], xcols[1 - p], semcs[1 - p])

            l_prev, c0_prev = infl[p]
            c0_prev = pl.multiple_of(c0_prev, _CB)

            @pl.when(s >= 2)
            def _():
                pltpu.make_async_copy(
                    bufs[p], out_hbm.at[l_prev, pl.ds(c0_prev, _CB)], sems[p]
                ).wait()
                _clear(p)

            @pl.when(t < _NTASK)
            def _():
                pltpu.make_async_copy(xt_hbm.at[l], xcols[p], semcs[p]).wait()

            def _issue():
                _plant(p, c0)
                pltpu.async_copy(bufs[p], out_hbm.at[l, pl.ds(c0, _CB)],
                                 sems[p])
                return (l, c0)

            infl[p] = lax.cond(t < _NTASK, _issue, lambda: infl[p])
            l, cc = _advance(l, cc)
        return (l, cc, tuple(infl))

    zero2 = (jnp.int32(0), jnp.int32(0))
    _, _, infl = lax.fori_loop(
        0, _SLOTS // 2, _slot_body, (l_init, cc_init, (zero2, zero2))
    )

    for s in (_SLOTS - 2, _SLOTS - 1):
        t = wid + _NW * s
        l_d, c0_d = infl[s % 2]
        c0_d = pl.multiple_of(c0_d, _CB)

        @pl.when(t < _NTASK)
        def _():
            pltpu.make_async_copy(
                bufs[s % 2], out_hbm.at[l_d, pl.ds(c0_d, _CB)], sems[s % 2]
            ).wait()


def kernel(x):
    xt = x.astype(jnp.int32).T
    out_t = _sc_onehot_t(xt)
    return jnp.transpose(out_t, (2, 0, 1))
